# baseline (speedup 1.0000x reference)
import functools

import jax
import jax.numpy as jnp
from jax import lax
from jax.experimental import pallas as pl
from jax.experimental.pallas import tpu as pltpu
from jax.experimental.pallas import tpu_sc as plsc

N_NODES = 10000
D = 128
H = 256
E_EDGES = 320000
NEXP = 4

NW = 32
ROWS_TOTAL = 2 * E_EDGES
RPW = ROWS_TOTAL // NW
CHUNK = 80
NCHUNK = RPW // CHUNK
KBUF = 5

BE = 5000
NB = E_EDGES // BE
WCOL = 640


def _sc_gather(z, idx3):
    mesh = plsc.VectorSubcoreMesh(core_axis_name="c", subcore_axis_name="s")

    @functools.partial(
        pl.kernel,
        mesh=mesh,
        out_type=jax.ShapeDtypeStruct((ROWS_TOTAL, D), jnp.float32),
        scratch_types=(
            [pltpu.VMEM((NCHUNK, CHUNK), jnp.int32)]
            + [pltpu.VMEM((CHUNK, D), jnp.float32) for _ in range(KBUF)]
            + [pltpu.SemaphoreType.DMA for _ in range(2 * KBUF)]
        ),
    )
    def gather_kernel(z_hbm, idx_hbm, out_hbm, idx_v, *rest):
        rows = rest[:KBUF]
        gsem = rest[KBUF:2 * KBUF]
        osem = rest[2 * KBUF:]
        wid = lax.axis_index("s") * 2 + lax.axis_index("c")
        base = wid * RPW
        pltpu.sync_copy(idx_hbm.at[wid], idx_v)

        def outer(j, _):
            c0 = j * KBUF
            handles = []
            for s in range(KBUF):
                @pl.when(j > 0)
                def _wait_out(s=s):
                    pltpu.make_async_copy(
                        rows[s], out_hbm.at[pl.ds(0, CHUNK)], osem[s]
                    ).wait()
                handles.append(
                    pltpu.async_copy(z_hbm.at[idx_v.at[c0 + s]], rows[s], gsem[s])
                )
            for s in range(KBUF):
                handles[s].wait()
                pltpu.async_copy(
                    rows[s],
                    out_hbm.at[pl.ds(base + (c0 + s) * CHUNK, CHUNK)],
                    osem[s],
                )
            return ()

        lax.fori_loop(0, NCHUNK // KBUF, outer, (), unroll=False)
        for s in range(KBUF):
            pltpu.make_async_copy(
                rows[s], out_hbm.at[pl.ds(0, CHUNK)], osem[s]
            ).wait()

    return gather_kernel(z, idx3)


def _tc_body(zu_ref, zv_ref, w1_ref, b1_ref, gb_ref, w2_ref, b2_ref,
             out_ref, aux_ref, acc_ref):
    i = pl.program_id(0)
    f32 = jnp.float32
    zu = zu_ref[...]
    zv = zv_ref[...]
    dd = jnp.abs(zu - zv)
    mm = zu * zv
    Tu = jnp.dot(zu, w1_ref[0], preferred_element_type=f32)
    Tv = jnp.dot(zv, w1_ref[1], preferred_element_type=f32)
    Td = jnp.dot(dd, w1_ref[2], preferred_element_type=f32)
    Tm = jnp.dot(mm, w1_ref[3], preferred_element_type=f32)
    b1 = b1_ref[...]
    h1 = jnp.maximum(Tu[:, 0:H] + Tv[:, 0:H] + b1[0], 0.0)
    h2 = jnp.maximum(Td[:, 0:H] + b1[1], 0.0)
    h3 = jnp.maximum(Tm[:, 0:H] + b1[2], 0.0)
    h4 = jnp.maximum(
        (Tu[:, H:2 * H] + Tv[:, H:2 * H]) + (Td[:, H:2 * H] + Tm[:, H:2 * H])
        + b1[3], 0.0)
    gl = ((Tu[:, 2 * H:] + Tv[:, 2 * H:]) + (Td[:, 2 * H:] + Tm[:, 2 * H:])
          + gb_ref[...])
    gmax = jnp.max(gl, axis=1, keepdims=True)
    ge = jnp.exp(gl - gmax)
    gsum = jnp.sum(ge, axis=1, keepdims=True)
    pmax = 1.0 / gsum
    b2 = b2_ref[...]
    s1 = jnp.dot(h1, w2_ref[0], preferred_element_type=f32)[:, 0:1] + b2[0:1]
    s2 = jnp.dot(h2, w2_ref[1], preferred_element_type=f32)[:, 0:1] + b2[1:2]
    s3 = jnp.dot(h3, w2_ref[2], preferred_element_type=f32)[:, 0:1] + b2[2:3]
    s4 = jnp.dot(h4, w2_ref[3], preferred_element_type=f32)[:, 0:1] + b2[3:4]
    c0 = gl[:, 0:1] >= gmax
    c1 = gl[:, 1:2] >= gmax
    c2 = gl[:, 2:3] >= gmax
    sel = jnp.where(c0, s1, jnp.where(c1, s2, jnp.where(c2, s3, s4)))
    out_ref[...] = pmax * sel

    @pl.when(i == 0)
    def _init():
        acc_ref[...] = jnp.zeros_like(acc_ref)

    acc_ref[...] += jnp.sum(ge * pmax, axis=0, keepdims=True)

    @pl.when(i == NB - 1)
    def _finish():
        avg = acc_ref[...] / float(E_EDGES)
        aux_ref[...] = (jnp.sum(avg * avg) * float(NEXP)).reshape(1, 1)


def _tc_moe(gathered, W1p, b1p, gbp, W2p, b2p):
    full = lambda shape: pl.BlockSpec(shape, lambda i: tuple(0 for _ in shape))
    return pl.pallas_call(
        _tc_body,
        grid=(NB,),
        in_specs=[
            pl.BlockSpec((BE, D), lambda i: (i, 0)),
            pl.BlockSpec((BE, D), lambda i: (i + NB, 0)),
            full((NEXP, D, WCOL)),
            full((NEXP, H)),
            full((1, 128)),
            full((NEXP, H, 128)),
            full((NEXP, 1)),
        ],
        out_specs=[
            pl.BlockSpec((BE, 1), lambda i: (i, 0)),
            pl.BlockSpec((1, 1), lambda i: (0, 0)),
        ],
        out_shape=[
            jax.ShapeDtypeStruct((E_EDGES, 1), jnp.float32),
            jax.ShapeDtypeStruct((1, 1), jnp.float32),
        ],
        scratch_shapes=[pltpu.VMEM((1, 128), jnp.float32)],
    )(gathered, gathered, W1p, b1p, gbp, W2p, b2p)


def kernel(g, z, uv, gate_W, gate_b, ec_W1, ec_b1, ec_W2, ec_b2,
           ed_W1, ed_b1, ed_W2, ed_b2, em_W1, em_b1, em_W2, em_b2,
           ea_W1, ea_b1, ea_W2, ea_b2):
    idx3 = uv.reshape(NW, NCHUNK, CHUNK)
    gathered = _sc_gather(z, idx3)

    gz = jnp.zeros((D, 128 - NEXP), jnp.float32)
    gcol = lambda c: jnp.concatenate([gate_W[c * D:(c + 1) * D], gz], axis=1)
    W1p = jnp.stack([
        jnp.concatenate([ec_W1[:D], ea_W1[:D], gcol(0)], axis=1),
        jnp.concatenate([ec_W1[D:], ea_W1[D:2 * D], gcol(1)], axis=1),
        jnp.concatenate([ed_W1, ea_W1[2 * D:3 * D], gcol(2)], axis=1),
        jnp.concatenate([em_W1, ea_W1[3 * D:], gcol(3)], axis=1),
    ])
    b1p = jnp.stack([ec_b1, ed_b1, em_b1, ea_b1])
    gbp = jnp.concatenate(
        [gate_b, jnp.full((128 - NEXP,), -1e30, jnp.float32)])[None, :]
    W2p = jnp.pad(jnp.stack([ec_W2, ed_W2, em_W2, ea_W2]),
                  ((0, 0), (0, 0), (0, 127)))
    b2p = jnp.stack([ec_b2, ed_b2, em_b2, ea_b2])

    out, aux = _tc_moe(gathered, W1p, b1p, gbp, W2p, b2p)
    return out, aux[0, 0]


# speedup vs baseline: 1.1191x; 1.1191x over previous
import functools

import jax
import jax.numpy as jnp
from jax import lax
from jax.experimental import pallas as pl
from jax.experimental.pallas import tpu as pltpu
from jax.experimental.pallas import tpu_sc as plsc

N_NODES = 10000
D = 128
H = 256
E_EDGES = 320000
NEXP = 4

NHALF = 2
E2 = E_EDGES // NHALF

NW = 32
ROWS_H = 2 * E2
RPW = ROWS_H // NW
CHUNK = 80
NCHUNK = RPW // CHUNK
KBUF = 5

BE = 3200
NB = E2 // BE
WCOL = 640


def _sc_gather(z, idx3):
    mesh = plsc.VectorSubcoreMesh(core_axis_name="c", subcore_axis_name="s")

    @functools.partial(
        pl.kernel,
        mesh=mesh,
        out_type=jax.ShapeDtypeStruct((ROWS_H, D), jnp.float32),
        scratch_types=(
            [pltpu.VMEM((NCHUNK, CHUNK), jnp.int32)]
            + [pltpu.VMEM((CHUNK, D), jnp.float32) for _ in range(KBUF)]
            + [pltpu.SemaphoreType.DMA for _ in range(2 * KBUF)]
        ),
    )
    def gather_kernel(z_hbm, idx_hbm, out_hbm, idx_v, *rest):
        rows = rest[:KBUF]
        gsem = rest[KBUF:2 * KBUF]
        osem = rest[2 * KBUF:]
        wid = lax.axis_index("s") * 2 + lax.axis_index("c")
        base = wid * RPW
        pltpu.sync_copy(idx_hbm.at[wid], idx_v)

        def outer(j, _):
            c0 = j * KBUF
            handles = []
            for s in range(KBUF):
                @pl.when(j > 0)
                def _wait_out(s=s):
                    pltpu.make_async_copy(
                        rows[s], out_hbm.at[pl.ds(0, CHUNK)], osem[s]
                    ).wait()
                handles.append(
                    pltpu.async_copy(z_hbm.at[idx_v.at[c0 + s]], rows[s], gsem[s])
                )
            for s in range(KBUF):
                handles[s].wait()
                pltpu.async_copy(
                    rows[s],
                    out_hbm.at[pl.ds(base + (c0 + s) * CHUNK, CHUNK)],
                    osem[s],
                )
            return ()

        lax.fori_loop(0, NCHUNK // KBUF, outer, (), unroll=False)
        for s in range(KBUF):
            pltpu.make_async_copy(
                rows[s], out_hbm.at[pl.ds(0, CHUNK)], osem[s]
            ).wait()

    return gather_kernel(z, idx3)


def _tc_body(zu_ref, zv_ref, w1_ref, b1_ref, gb_ref, w2_ref, b2_ref,
             accin_ref, out_ref, acc_ref, aux_ref):
    i = pl.program_id(0)
    f32 = jnp.float32
    zu = zu_ref[...]
    zv = zv_ref[...]
    dd = jnp.abs(zu - zv)
    mm = zu * zv
    Tu = jnp.dot(zu, w1_ref[0], preferred_element_type=f32)
    Tv = jnp.dot(zv, w1_ref[1], preferred_element_type=f32)
    Td = jnp.dot(dd, w1_ref[2], preferred_element_type=f32)
    Tm = jnp.dot(mm, w1_ref[3], preferred_element_type=f32)
    b1 = b1_ref[...]
    h1 = jnp.maximum(Tu[:, 0:H] + Tv[:, 0:H] + b1[0], 0.0)
    h2 = jnp.maximum(Td[:, 0:H] + b1[1], 0.0)
    h3 = jnp.maximum(Tm[:, 0:H] + b1[2], 0.0)
    h4 = jnp.maximum(
        (Tu[:, H:2 * H] + Tv[:, H:2 * H]) + (Td[:, H:2 * H] + Tm[:, H:2 * H])
        + b1[3], 0.0)
    gl = ((Tu[:, 2 * H:] + Tv[:, 2 * H:]) + (Td[:, 2 * H:] + Tm[:, 2 * H:])
          + gb_ref[...])
    gmax = jnp.max(gl, axis=1, keepdims=True)
    ge = jnp.exp(gl - gmax)
    gsum = jnp.sum(ge, axis=1, keepdims=True)
    pmax = 1.0 / gsum
    b2 = b2_ref[...]
    s1 = jnp.dot(h1, w2_ref[0], preferred_element_type=f32)[:, 0:1] + b2[0:1]
    s2 = jnp.dot(h2, w2_ref[1], preferred_element_type=f32)[:, 0:1] + b2[1:2]
    s3 = jnp.dot(h3, w2_ref[2], preferred_element_type=f32)[:, 0:1] + b2[2:3]
    s4 = jnp.dot(h4, w2_ref[3], preferred_element_type=f32)[:, 0:1] + b2[3:4]
    c0 = gl[:, 0:1] >= gmax
    c1 = gl[:, 1:2] >= gmax
    c2 = gl[:, 2:3] >= gmax
    sel = jnp.where(c0, s1, jnp.where(c1, s2, jnp.where(c2, s3, s4)))
    out_ref[...] = pmax * sel

    @pl.when(i == 0)
    def _init():
        acc_ref[...] = accin_ref[...]

    acc_ref[...] += jnp.sum(ge * pmax, axis=0, keepdims=True)

    @pl.when(i == NB - 1)
    def _finish():
        avg = acc_ref[...] / float(E_EDGES)
        aux_ref[...] = (jnp.sum(avg * avg) * float(NEXP)).reshape(1, 1)


def _tc_moe(gathered, accin, W1p, b1p, gbp, W2p, b2p):
    full = lambda shape: pl.BlockSpec(shape, lambda i: tuple(0 for _ in shape))
    return pl.pallas_call(
        _tc_body,
        grid=(NB,),
        in_specs=[
            pl.BlockSpec((BE, D), lambda i: (i, 0)),
            pl.BlockSpec((BE, D), lambda i: (i + NB, 0)),
            full((NEXP, D, WCOL)),
            full((NEXP, H)),
            full((1, 128)),
            full((NEXP, H, 128)),
            full((NEXP, 1)),
            full((1, 128)),
        ],
        out_specs=[
            pl.BlockSpec((BE, 1), lambda i: (i, 0)),
            pl.BlockSpec((1, 128), lambda i: (0, 0)),
            pl.BlockSpec((1, 1), lambda i: (0, 0)),
        ],
        out_shape=[
            jax.ShapeDtypeStruct((E2, 1), jnp.float32),
            jax.ShapeDtypeStruct((1, 128), jnp.float32),
            jax.ShapeDtypeStruct((1, 1), jnp.float32),
        ],
    )(gathered, gathered, W1p, b1p, gbp, W2p, b2p, accin)


def kernel(g, z, uv, gate_W, gate_b, ec_W1, ec_b1, ec_W2, ec_b2,
           ed_W1, ed_b1, ed_W2, ed_b2, em_W1, em_b1, em_W2, em_b2,
           ea_W1, ea_b1, ea_W2, ea_b2):
    g1 = _sc_gather(z, uv[:, :E2].reshape(NW, NCHUNK, CHUNK))
    g2 = _sc_gather(z, uv[:, E2:].reshape(NW, NCHUNK, CHUNK))

    gz = jnp.zeros((D, 128 - NEXP), jnp.float32)
    gcol = lambda c: jnp.concatenate([gate_W[c * D:(c + 1) * D], gz], axis=1)
    W1p = jnp.stack([
        jnp.concatenate([ec_W1[:D], ea_W1[:D], gcol(0)], axis=1),
        jnp.concatenate([ec_W1[D:], ea_W1[D:2 * D], gcol(1)], axis=1),
        jnp.concatenate([ed_W1, ea_W1[2 * D:3 * D], gcol(2)], axis=1),
        jnp.concatenate([em_W1, ea_W1[3 * D:], gcol(3)], axis=1),
    ])
    b1p = jnp.stack([ec_b1, ed_b1, em_b1, ea_b1])
    gbp = jnp.concatenate(
        [gate_b, jnp.full((128 - NEXP,), -1e30, jnp.float32)])[None, :]
    W2p = jnp.pad(jnp.stack([ec_W2, ed_W2, em_W2, ea_W2]),
                  ((0, 0), (0, 0), (0, 127)))
    b2p = jnp.stack([ec_b2, ed_b2, em_b2, ea_b2])

    acc0 = jnp.zeros((1, 128), jnp.float32)
    out1, acc1, _ = _tc_moe(g1, acc0, W1p, b1p, gbp, W2p, b2p)
    out2, _, aux = _tc_moe(g2, acc1, W1p, b1p, gbp, W2p, b2p)
    return jnp.concatenate([out1, out2], axis=0), aux[0, 0]


# speedup vs baseline: 1.1348x; 1.0140x over previous
import functools

import jax
import jax.numpy as jnp
from jax import lax
from jax.experimental import pallas as pl
from jax.experimental.pallas import tpu as pltpu
from jax.experimental.pallas import tpu_sc as plsc

N_NODES = 10000
D = 128
H = 256
E_EDGES = 320000
NEXP = 4

NHALF = 2
E2 = E_EDGES // NHALF

NW = 32
ROWS_H = 2 * E2
RPW = ROWS_H // NW
CHUNK = 80
NCHUNK = RPW // CHUNK
KBUF = 5

BE = 4000
NB = E2 // BE
WCOL = 640


def _sc_gather(z, idx3):
    mesh = plsc.VectorSubcoreMesh(core_axis_name="c", subcore_axis_name="s")

    @functools.partial(
        pl.kernel,
        mesh=mesh,
        out_type=jax.ShapeDtypeStruct((ROWS_H, D), jnp.float32),
        scratch_types=(
            [pltpu.VMEM((NCHUNK, CHUNK), jnp.int32)]
            + [pltpu.VMEM((CHUNK, D), jnp.float32) for _ in range(KBUF)]
            + [pltpu.SemaphoreType.DMA for _ in range(2 * KBUF)]
        ),
    )
    def gather_kernel(z_hbm, idx_hbm, out_hbm, idx_v, *rest):
        rows = rest[:KBUF]
        gsem = rest[KBUF:2 * KBUF]
        osem = rest[2 * KBUF:]
        wid = lax.axis_index("s") * 2 + lax.axis_index("c")
        base = wid * RPW
        pltpu.sync_copy(idx_hbm.at[wid], idx_v)

        def outer(j, _):
            c0 = j * KBUF
            handles = []
            for s in range(KBUF):
                @pl.when(j > 0)
                def _wait_out(s=s):
                    pltpu.make_async_copy(
                        rows[s], out_hbm.at[pl.ds(0, CHUNK)], osem[s]
                    ).wait()
                handles.append(
                    pltpu.async_copy(z_hbm.at[idx_v.at[c0 + s]], rows[s], gsem[s])
                )
            for s in range(KBUF):
                handles[s].wait()
                pltpu.async_copy(
                    rows[s],
                    out_hbm.at[pl.ds(base + (c0 + s) * CHUNK, CHUNK)],
                    osem[s],
                )
            return ()

        lax.fori_loop(0, NCHUNK // KBUF, outer, (), unroll=False)
        for s in range(KBUF):
            pltpu.make_async_copy(
                rows[s], out_hbm.at[pl.ds(0, CHUNK)], osem[s]
            ).wait()

    return gather_kernel(z, idx3)


def _tc_body(zu_ref, zv_ref, w1_ref, b1_ref, gb_ref, w2_ref, b2_ref,
             accin_ref, out_ref, acc_ref, aux_ref):
    i = pl.program_id(0)
    f32 = jnp.float32
    zu = zu_ref[...]
    zv = zv_ref[...]
    dd = jnp.abs(zu - zv)
    mm = zu * zv
    Tu = jnp.dot(zu, w1_ref[0], preferred_element_type=f32)
    Tv = jnp.dot(zv, w1_ref[1], preferred_element_type=f32)
    Td = jnp.dot(dd, w1_ref[2], preferred_element_type=f32)
    Tm = jnp.dot(mm, w1_ref[3], preferred_element_type=f32)
    b1 = b1_ref[...]
    h1 = jnp.maximum(Tu[:, 0:H] + Tv[:, 0:H] + b1[0], 0.0)
    h2 = jnp.maximum(Td[:, 0:H] + b1[1], 0.0)
    h3 = jnp.maximum(Tm[:, 0:H] + b1[2], 0.0)
    h4 = jnp.maximum(
        (Tu[:, H:2 * H] + Tv[:, H:2 * H]) + (Td[:, H:2 * H] + Tm[:, H:2 * H])
        + b1[3], 0.0)
    gl = ((Tu[:, 2 * H:] + Tv[:, 2 * H:]) + (Td[:, 2 * H:] + Tm[:, 2 * H:])
          + gb_ref[...])
    gmax = jnp.max(gl, axis=1, keepdims=True)
    ge = jnp.exp(gl - gmax)
    gsum = jnp.sum(ge, axis=1, keepdims=True)
    pmax = 1.0 / gsum
    b2 = b2_ref[...]
    s1 = jnp.dot(h1, w2_ref[0], preferred_element_type=f32)[:, 0:1] + b2[0:1]
    s2 = jnp.dot(h2, w2_ref[1], preferred_element_type=f32)[:, 0:1] + b2[1:2]
    s3 = jnp.dot(h3, w2_ref[2], preferred_element_type=f32)[:, 0:1] + b2[2:3]
    s4 = jnp.dot(h4, w2_ref[3], preferred_element_type=f32)[:, 0:1] + b2[3:4]
    c0 = gl[:, 0:1] >= gmax
    c1 = gl[:, 1:2] >= gmax
    c2 = gl[:, 2:3] >= gmax
    sel = jnp.where(c0, s1, jnp.where(c1, s2, jnp.where(c2, s3, s4)))
    out_ref[...] = pmax * sel

    @pl.when(i == 0)
    def _init():
        acc_ref[...] = accin_ref[...]

    acc_ref[...] += jnp.sum(ge * pmax, axis=0, keepdims=True)

    @pl.when(i == NB - 1)
    def _finish():
        avg = acc_ref[...] / float(E_EDGES)
        aux_ref[...] = (jnp.sum(avg * avg) * float(NEXP)).reshape(1, 1)


def _tc_moe(gathered, accin, W1p, b1p, gbp, W2p, b2p):
    full = lambda shape: pl.BlockSpec(shape, lambda i: tuple(0 for _ in shape))
    return pl.pallas_call(
        _tc_body,
        grid=(NB,),
        in_specs=[
            pl.BlockSpec((BE, D), lambda i: (i, 0)),
            pl.BlockSpec((BE, D), lambda i: (i + NB, 0)),
            full((NEXP, D, WCOL)),
            full((NEXP, H)),
            full((1, 128)),
            full((NEXP, H, 128)),
            full((NEXP, 1)),
            full((1, 128)),
        ],
        out_specs=[
            pl.BlockSpec((BE, 1), lambda i: (i, 0)),
            pl.BlockSpec((1, 128), lambda i: (0, 0)),
            pl.BlockSpec((1, 1), lambda i: (0, 0)),
        ],
        out_shape=[
            jax.ShapeDtypeStruct((E2, 1), jnp.float32),
            jax.ShapeDtypeStruct((1, 128), jnp.float32),
            jax.ShapeDtypeStruct((1, 1), jnp.float32),
        ],
    )(gathered, gathered, W1p, b1p, gbp, W2p, b2p, accin)


def kernel(g, z, uv, gate_W, gate_b, ec_W1, ec_b1, ec_W2, ec_b2,
           ed_W1, ed_b1, ed_W2, ed_b2, em_W1, em_b1, em_W2, em_b2,
           ea_W1, ea_b1, ea_W2, ea_b2):
    g1 = _sc_gather(z, uv[:, :E2].reshape(NW, NCHUNK, CHUNK))
    g2 = _sc_gather(z, uv[:, E2:].reshape(NW, NCHUNK, CHUNK))

    gz = jnp.zeros((D, 128 - NEXP), jnp.float32)
    gcol = lambda c: jnp.concatenate([gate_W[c * D:(c + 1) * D], gz], axis=1)
    W1p = jnp.stack([
        jnp.concatenate([ec_W1[:D], ea_W1[:D], gcol(0)], axis=1),
        jnp.concatenate([ec_W1[D:], ea_W1[D:2 * D], gcol(1)], axis=1),
        jnp.concatenate([ed_W1, ea_W1[2 * D:3 * D], gcol(2)], axis=1),
        jnp.concatenate([em_W1, ea_W1[3 * D:], gcol(3)], axis=1),
    ])
    b1p = jnp.stack([ec_b1, ed_b1, em_b1, ea_b1])
    gbp = jnp.concatenate(
        [gate_b, jnp.full((128 - NEXP,), -1e30, jnp.float32)])[None, :]
    W2p = jnp.pad(jnp.stack([ec_W2, ed_W2, em_W2, ea_W2]),
                  ((0, 0), (0, 0), (0, 127)))
    b2p = jnp.stack([ec_b2, ed_b2, em_b2, ea_b2])

    acc0 = jnp.zeros((1, 128), jnp.float32)
    out1, acc1, _ = _tc_moe(g1, acc0, W1p, b1p, gbp, W2p, b2p)
    out2, _, aux = _tc_moe(g2, acc1, W1p, b1p, gbp, W2p, b2p)
    return jnp.concatenate([out1, out2], axis=0), aux[0, 0]


# speedup vs baseline: 1.1933x; 1.0515x over previous
import functools

import jax
import jax.numpy as jnp
from jax import lax
from jax.experimental import pallas as pl
from jax.experimental.pallas import tpu as pltpu
from jax.experimental.pallas import tpu_sc as plsc

N_NODES = 10000
D = 128
H = 256
E_EDGES = 320000
NEXP = 4

E_A = 80000
E_B = E_EDGES - E_A
CHUNK_A = 40
CHUNK_B = 120

NW = 32
KBUF = 5

BE = 4000
WCOL = 640


def _sc_gather(z, idx3, chunk):
    nchunk = idx3.shape[1]
    rpw = nchunk * chunk
    rows_total = NW * rpw
    mesh = plsc.VectorSubcoreMesh(core_axis_name="c", subcore_axis_name="s")

    @functools.partial(
        pl.kernel,
        mesh=mesh,
        out_type=jax.ShapeDtypeStruct((rows_total, D), jnp.float32),
        scratch_types=(
            [pltpu.VMEM((nchunk, chunk), jnp.int32)]
            + [pltpu.VMEM((chunk, D), jnp.float32) for _ in range(KBUF)]
            + [pltpu.SemaphoreType.DMA for _ in range(2 * KBUF)]
        ),
    )
    def gather_kernel(z_hbm, idx_hbm, out_hbm, idx_v, *rest):
        rows = rest[:KBUF]
        gsem = rest[KBUF:2 * KBUF]
        osem = rest[2 * KBUF:]
        wid = lax.axis_index("s") * 2 + lax.axis_index("c")
        base = wid * rpw
        pltpu.sync_copy(idx_hbm.at[wid], idx_v)

        def outer(j, _):
            c0 = j * KBUF
            handles = []
            for s in range(KBUF):
                @pl.when(j > 0)
                def _wait_out(s=s):
                    pltpu.make_async_copy(
                        rows[s], out_hbm.at[pl.ds(0, chunk)], osem[s]
                    ).wait()
                handles.append(
                    pltpu.async_copy(z_hbm.at[idx_v.at[c0 + s]], rows[s], gsem[s])
                )
            for s in range(KBUF):
                handles[s].wait()
                pltpu.async_copy(
                    rows[s],
                    out_hbm.at[pl.ds(base + (c0 + s) * chunk, chunk)],
                    osem[s],
                )
            return ()

        lax.fori_loop(0, nchunk // KBUF, outer, (), unroll=False)
        for s in range(KBUF):
            pltpu.make_async_copy(
                rows[s], out_hbm.at[pl.ds(0, chunk)], osem[s]
            ).wait()

    return gather_kernel(z, idx3)


def _tc_body(nb, zu_ref, zv_ref, w1_ref, b1_ref, gb_ref, w2_ref, b2_ref,
             accin_ref, out_ref, acc_ref, aux_ref):
    i = pl.program_id(0)
    f32 = jnp.float32
    zu = zu_ref[...]
    zv = zv_ref[...]
    dd = jnp.abs(zu - zv)
    mm = zu * zv
    Tu = jnp.dot(zu, w1_ref[0], preferred_element_type=f32)
    Tv = jnp.dot(zv, w1_ref[1], preferred_element_type=f32)
    Td = jnp.dot(dd, w1_ref[2], preferred_element_type=f32)
    Tm = jnp.dot(mm, w1_ref[3], preferred_element_type=f32)
    b1 = b1_ref[...]
    h1 = jnp.maximum(Tu[:, 0:H] + Tv[:, 0:H] + b1[0], 0.0)
    h2 = jnp.maximum(Td[:, 0:H] + b1[1], 0.0)
    h3 = jnp.maximum(Tm[:, 0:H] + b1[2], 0.0)
    h4 = jnp.maximum(
        (Tu[:, H:2 * H] + Tv[:, H:2 * H]) + (Td[:, H:2 * H] + Tm[:, H:2 * H])
        + b1[3], 0.0)
    gl = ((Tu[:, 2 * H:] + Tv[:, 2 * H:]) + (Td[:, 2 * H:] + Tm[:, 2 * H:])
          + gb_ref[...])
    gmax = jnp.max(gl, axis=1, keepdims=True)
    ge = jnp.exp(gl - gmax)
    gsum = jnp.sum(ge, axis=1, keepdims=True)
    pmax = 1.0 / gsum
    b2 = b2_ref[...]
    s1 = jnp.dot(h1, w2_ref[0], preferred_element_type=f32)[:, 0:1] + b2[0:1]
    s2 = jnp.dot(h2, w2_ref[1], preferred_element_type=f32)[:, 0:1] + b2[1:2]
    s3 = jnp.dot(h3, w2_ref[2], preferred_element_type=f32)[:, 0:1] + b2[2:3]
    s4 = jnp.dot(h4, w2_ref[3], preferred_element_type=f32)[:, 0:1] + b2[3:4]
    c0 = gl[:, 0:1] >= gmax
    c1 = gl[:, 1:2] >= gmax
    c2 = gl[:, 2:3] >= gmax
    sel = jnp.where(c0, s1, jnp.where(c1, s2, jnp.where(c2, s3, s4)))
    out_ref[...] = pmax * sel

    @pl.when(i == 0)
    def _init():
        acc_ref[...] = accin_ref[...]

    acc_ref[...] += jnp.sum(ge * pmax, axis=0, keepdims=True)

    @pl.when(i == nb - 1)
    def _finish():
        avg = acc_ref[...] / float(E_EDGES)
        aux_ref[...] = (jnp.sum(avg * avg) * float(NEXP)).reshape(1, 1)


def _tc_moe(gathered, accin, W1p, b1p, gbp, W2p, b2p, ne):
    nb = ne // BE
    full = lambda shape: pl.BlockSpec(shape, lambda i: tuple(0 for _ in shape))
    return pl.pallas_call(
        functools.partial(_tc_body, nb),
        grid=(nb,),
        in_specs=[
            pl.BlockSpec((BE, D), lambda i: (i, 0)),
            pl.BlockSpec((BE, D), lambda i: (i + nb, 0)),
            full((NEXP, D, WCOL)),
            full((NEXP, H)),
            full((1, 128)),
            full((NEXP, H, 128)),
            full((NEXP, 1)),
            full((1, 128)),
        ],
        out_specs=[
            pl.BlockSpec((BE, 1), lambda i: (i, 0)),
            pl.BlockSpec((1, 128), lambda i: (0, 0)),
            pl.BlockSpec((1, 1), lambda i: (0, 0)),
        ],
        out_shape=[
            jax.ShapeDtypeStruct((ne, 1), jnp.float32),
            jax.ShapeDtypeStruct((1, 128), jnp.float32),
            jax.ShapeDtypeStruct((1, 1), jnp.float32),
        ],
    )(gathered, gathered, W1p, b1p, gbp, W2p, b2p, accin)


def kernel(g, z, uv, gate_W, gate_b, ec_W1, ec_b1, ec_W2, ec_b2,
           ed_W1, ed_b1, ed_W2, ed_b2, em_W1, em_b1, em_W2, em_b2,
           ea_W1, ea_b1, ea_W2, ea_b2):
    g1 = _sc_gather(z, uv[:, :E_A].reshape(NW, -1, CHUNK_A), CHUNK_A)
    g2 = _sc_gather(z, uv[:, E_A:].reshape(NW, -1, CHUNK_B), CHUNK_B)

    gz = jnp.zeros((D, 128 - NEXP), jnp.float32)
    gcol = lambda c: jnp.concatenate([gate_W[c * D:(c + 1) * D], gz], axis=1)
    W1p = jnp.stack([
        jnp.concatenate([ec_W1[:D], ea_W1[:D], gcol(0)], axis=1),
        jnp.concatenate([ec_W1[D:], ea_W1[D:2 * D], gcol(1)], axis=1),
        jnp.concatenate([ed_W1, ea_W1[2 * D:3 * D], gcol(2)], axis=1),
        jnp.concatenate([em_W1, ea_W1[3 * D:], gcol(3)], axis=1),
    ])
    b1p = jnp.stack([ec_b1, ed_b1, em_b1, ea_b1])
    gbp = jnp.concatenate(
        [gate_b, jnp.full((128 - NEXP,), -1e30, jnp.float32)])[None, :]
    W2p = jnp.pad(jnp.stack([ec_W2, ed_W2, em_W2, ea_W2]),
                  ((0, 0), (0, 0), (0, 127)))
    b2p = jnp.stack([ec_b2, ed_b2, em_b2, ea_b2])

    acc0 = jnp.zeros((1, 128), jnp.float32)
    out1, acc1, _ = _tc_moe(g1, acc0, W1p, b1p, gbp, W2p, b2p, E_A)
    out2, _, aux = _tc_moe(g2, acc1, W1p, b1p, gbp, W2p, b2p, E_B)
    return jnp.concatenate([out1, out2], axis=0), aux[0, 0]
